# baseline (speedup 1.0000x reference)
import functools

import jax
import jax.numpy as jnp
from jax import lax
from jax.experimental import pallas as pl
from jax.experimental.pallas import tpu as pltpu
from jax.experimental.pallas import tpu_sc as plsc

DIM = 64
NC = 2
NS = 16
NW = NC * NS
CHUNK = 128


def _make_gather(npos, nblk):
    assert nblk == NW
    mesh = plsc.VectorSubcoreMesh(core_axis_name="c", subcore_axis_name="s")

    @functools.partial(
        pl.kernel,
        mesh=mesh,
        compiler_params=pltpu.CompilerParams(use_tc_tiling_on_sc=False,
                                             needs_layout_passes=False),
        out_type=jax.ShapeDtypeStruct((npos, DIM, nblk * CHUNK), jnp.float32),
        scratch_types=[
            pltpu.VMEM((npos, CHUNK), jnp.int32),
            pltpu.VMEM((CHUNK, DIM), jnp.float32),
            pltpu.VMEM((CHUNK, DIM), jnp.float32),
            pltpu.VMEM((DIM, CHUNK), jnp.float32),
            pltpu.VMEM((DIM, CHUNK), jnp.float32),
            pltpu.SemaphoreType.DMA,
            pltpu.SemaphoreType.DMA,
            pltpu.SemaphoreType.DMA,
            pltpu.SemaphoreType.DMA,
        ],
    )
    def gather_kernel(idx_hbm, table_hbm, out_hbm,
                      idx_v, rows0, rows1, tb0, tb1, g0, g1, s0, s1):
        wid = lax.axis_index("s") * NC + lax.axis_index("c")
        rows = (rows0, rows1)
        tb = (tb0, tb1)
        gsem = (g0, g1)
        ssem = (s0, s1)
        iota = lax.iota(jnp.int32, 16)
        pltpu.sync_copy(idx_hbm.at[:, wid], idx_v)
        pltpu.async_copy(table_hbm.at[idx_v.at[0]], rows[0], gsem[0])

        def transpose_block(rbuf, obuf):
            def col(c, carry):
                cvec = jnp.full((16,), 0, jnp.int32) + c
                for k in range(CHUNK // 16):
                    vals = plsc.load_gather(rbuf, [k * 16 + iota, cvec])
                    obuf[c, pl.ds(k * 16, 16)] = vals
                return carry
            lax.fori_loop(0, DIM, col, 0)

        def outer(j, carry):
            for b in range(2):
                cur = 2 * j + b
                nb = 1 - b
                @pl.when(cur + 1 < npos)
                def _():
                    pltpu.async_copy(table_hbm.at[idx_v.at[cur + 1]],
                                     rows[nb], gsem[nb])
                pltpu.make_async_copy(table_hbm.at[idx_v.at[cur]],
                                      rows[b], gsem[b]).wait()
                @pl.when(cur >= 2)
                def _():
                    pltpu.make_async_copy(
                        tb[b], out_hbm.at[0, :, pl.ds(0, CHUNK)],
                        ssem[b]).wait()
                transpose_block(rows[b], tb[b])
                pltpu.async_copy(tb[b],
                                 out_hbm.at[cur, :, pl.ds(wid * CHUNK, CHUNK)],
                                 ssem[b])
            return carry

        lax.fori_loop(0, npos // 2, outer, 0)
        for b in range(2):
            pltpu.make_async_copy(tb[b], out_hbm.at[0, :, pl.ds(0, CHUNK)],
                                  ssem[b]).wait()

    return gather_kernel


def kernel(x, weight):
    batch, npos = x.shape
    assert batch % CHUNK == 0 and batch // CHUNK == NW
    idx = x.T.reshape(npos, NW, CHUNK).astype(jnp.int32)
    out_t = _make_gather(npos, NW)(idx, weight)
    return out_t.transpose(2, 0, 1)


# speedup vs baseline: 1.2607x; 1.2607x over previous
import functools

import jax
import jax.numpy as jnp
from jax import lax
from jax.experimental import pallas as pl
from jax.experimental.pallas import tpu as pltpu
from jax.experimental.pallas import tpu_sc as plsc

DIM = 64
NC = 2
NS = 16
NW = NC * NS
CHUNK = 128


def _make_gather(npos, batch):
    assert batch == NW * CHUNK
    assert npos % 2 == 0
    mesh = plsc.VectorSubcoreMesh(core_axis_name="c", subcore_axis_name="s")

    @functools.partial(
        pl.kernel,
        mesh=mesh,
        compiler_params=pltpu.CompilerParams(use_tc_tiling_on_sc=False),
        out_type=jax.ShapeDtypeStruct((npos * batch, DIM), jnp.float32),
        scratch_types=[
            pltpu.VMEM((npos, CHUNK), jnp.int32),
            pltpu.VMEM((2, CHUNK, DIM), jnp.float32),
            pltpu.SemaphoreType.DMA,
            pltpu.SemaphoreType.DMA,
            pltpu.SemaphoreType.DMA,
            pltpu.SemaphoreType.DMA,
        ],
    )
    def gather_kernel(idx_hbm, table_hbm, out_hbm,
                      idx_v, rows_v, g0, g1, s0, s1):
        wid = lax.axis_index("s") * NC + lax.axis_index("c")
        base = wid * CHUNK
        gsem = (g0, g1)
        ssem = (s0, s1)
        pltpu.sync_copy(idx_hbm.at[:, pl.ds(base, CHUNK)], idx_v)
        pltpu.async_copy(table_hbm.at[idx_v.at[0]], rows_v.at[0], gsem[0])

        def outer(j, carry):
            for b in range(2):
                cur = 2 * j + b
                nb = 1 - b
                @pl.when(cur + 1 < npos)
                def _():
                    @pl.when(cur >= 1)
                    def _():
                        pltpu.make_async_copy(
                            rows_v.at[nb],
                            out_hbm.at[pl.ds(0, CHUNK)],
                            ssem[nb]).wait()
                    pltpu.async_copy(table_hbm.at[idx_v.at[cur + 1]],
                                     rows_v.at[nb], gsem[nb])
                pltpu.make_async_copy(table_hbm.at[idx_v.at[cur]],
                                      rows_v.at[b], gsem[b]).wait()
                pltpu.async_copy(
                    rows_v.at[b],
                    out_hbm.at[pl.ds(cur * batch + base, CHUNK)],
                    ssem[b])
            return carry

        lax.fori_loop(0, npos // 2, outer, 0)
        for b in range(2):
            pltpu.make_async_copy(rows_v.at[b],
                                  out_hbm.at[pl.ds(0, CHUNK)],
                                  ssem[b]).wait()

    return gather_kernel


def kernel(x, weight):
    batch, npos = x.shape
    xt = x.T.astype(jnp.int32)
    out = _make_gather(npos, batch)(xt, weight)
    return out.reshape(npos, batch, DIM).transpose(1, 0, 2)


# speedup vs baseline: 1.3338x; 1.0579x over previous
import functools

import jax
import jax.numpy as jnp
from jax import lax
from jax.experimental import pallas as pl
from jax.experimental.pallas import tpu as pltpu
from jax.experimental.pallas import tpu_sc as plsc

DIM = 64
PAD = 128
NC = 2
NS = 16
NW = NC * NS
CHUNK = 128


def _make_gather(npos, batch):
    assert batch == NW * CHUNK
    assert npos % 2 == 0
    mesh = plsc.VectorSubcoreMesh(core_axis_name="c", subcore_axis_name="s")

    @functools.partial(
        pl.kernel,
        mesh=mesh,
        compiler_params=pltpu.CompilerParams(use_tc_tiling_on_sc=False),
        out_type=jax.ShapeDtypeStruct((npos * batch, DIM), jnp.float32),
        scratch_types=[
            pltpu.VMEM((npos, CHUNK), jnp.int32),
            pltpu.VMEM((2, CHUNK, PAD), jnp.float32),
            pltpu.SemaphoreType.DMA,
            pltpu.SemaphoreType.DMA,
            pltpu.SemaphoreType.DMA,
            pltpu.SemaphoreType.DMA,
        ],
    )
    def gather_kernel(idx_hbm, table_hbm, out_hbm,
                      idx_v, rows_v, g0, g1, s0, s1):
        wid = lax.axis_index("s") * NC + lax.axis_index("c")
        base = wid * CHUNK
        gsem = (g0, g1)
        ssem = (s0, s1)
        pltpu.sync_copy(idx_hbm.at[:, pl.ds(base, CHUNK)], idx_v)
        pltpu.async_copy(table_hbm.at[idx_v.at[0]], rows_v.at[0], gsem[0])

        def outer(j, carry):
            for b in range(2):
                cur = 2 * j + b
                nb = 1 - b
                @pl.when(cur + 1 < npos)
                def _():
                    @pl.when(cur >= 1)
                    def _():
                        pltpu.make_async_copy(
                            rows_v.at[nb, :, pl.ds(0, DIM)],
                            out_hbm.at[pl.ds(0, CHUNK)],
                            ssem[nb]).wait()
                    pltpu.async_copy(table_hbm.at[idx_v.at[cur + 1]],
                                     rows_v.at[nb], gsem[nb])
                pltpu.make_async_copy(table_hbm.at[idx_v.at[cur]],
                                      rows_v.at[b], gsem[b]).wait()
                pltpu.async_copy(
                    rows_v.at[b, :, pl.ds(0, DIM)],
                    out_hbm.at[pl.ds(cur * batch + base, CHUNK)],
                    ssem[b])
            return carry

        lax.fori_loop(0, npos // 2, outer, 0)
        for b in range(2):
            pltpu.make_async_copy(rows_v.at[b, :, pl.ds(0, DIM)],
                                  out_hbm.at[pl.ds(0, CHUNK)],
                                  ssem[b]).wait()

    return gather_kernel


def kernel(x, weight):
    batch, npos = x.shape
    xt = x.T.astype(jnp.int32)
    wpad = jnp.pad(weight, ((0, 0), (0, PAD - DIM)))
    out = _make_gather(npos, batch)(xt, wpad)
    return out.reshape(npos, batch, DIM).transpose(1, 0, 2)


# speedup vs baseline: 1.3747x; 1.0307x over previous
import functools

import jax
import jax.numpy as jnp
from jax import lax
from jax.experimental import pallas as pl
from jax.experimental.pallas import tpu as pltpu
from jax.experimental.pallas import tpu_sc as plsc

DIM = 64
PAD = 128
NC = 2
NS = 16
NW = NC * NS
CHUNK = 128


def _make_gather(npos, batch):
    assert batch == NW * CHUNK
    assert npos % 2 == 0
    mesh = plsc.VectorSubcoreMesh(core_axis_name="c", subcore_axis_name="s")

    @functools.partial(
        pl.kernel,
        mesh=mesh,
        compiler_params=pltpu.CompilerParams(use_tc_tiling_on_sc=False),
        out_type=jax.ShapeDtypeStruct((npos * batch, DIM), jnp.float32),
        scratch_types=[
            pltpu.VMEM((npos, CHUNK), jnp.int32),
            pltpu.VMEM((2, CHUNK, DIM), jnp.float32),
            pltpu.SemaphoreType.DMA,
            pltpu.SemaphoreType.DMA,
            pltpu.SemaphoreType.DMA,
            pltpu.SemaphoreType.DMA,
        ],
    )
    def gather_kernel(idx_hbm, table_hbm, out_hbm,
                      idx_v, rows_v, g0, g1, s0, s1):
        wid = lax.axis_index("s") * NC + lax.axis_index("c")
        base = wid * CHUNK
        gsem = (g0, g1)
        ssem = (s0, s1)
        pltpu.sync_copy(idx_hbm.at[:, pl.ds(base, CHUNK)], idx_v)

        def dbl(p, carry):
            for k in range(CHUNK // 16):
                v = idx_v[p, pl.ds(k * 16, 16)]
                idx_v[p, pl.ds(k * 16, 16)] = v + v
            return carry

        lax.fori_loop(0, npos, dbl, 0)
        pltpu.async_copy(table_hbm.at[idx_v.at[0]], rows_v.at[0], gsem[0])

        def outer(j, carry):
            for b in range(2):
                cur = 2 * j + b
                nb = 1 - b
                @pl.when(cur + 1 < npos)
                def _():
                    @pl.when(cur >= 1)
                    def _():
                        pltpu.make_async_copy(
                            rows_v.at[nb],
                            out_hbm.at[pl.ds(0, CHUNK)],
                            ssem[nb]).wait()
                    pltpu.async_copy(table_hbm.at[idx_v.at[cur + 1]],
                                     rows_v.at[nb], gsem[nb])
                pltpu.make_async_copy(table_hbm.at[idx_v.at[cur]],
                                      rows_v.at[b], gsem[b]).wait()
                pltpu.async_copy(
                    rows_v.at[b],
                    out_hbm.at[pl.ds(cur * batch + base, CHUNK)],
                    ssem[b])
            return carry

        lax.fori_loop(0, npos // 2, outer, 0)
        for b in range(2):
            pltpu.make_async_copy(rows_v.at[b],
                                  out_hbm.at[pl.ds(0, CHUNK)],
                                  ssem[b]).wait()

    return gather_kernel


def kernel(x, weight):
    batch, npos = x.shape
    xt = x.T.astype(jnp.int32)
    wpad = jnp.pad(weight, ((0, 0), (0, PAD - DIM)))
    w2 = wpad.reshape(2 * wpad.shape[0], DIM)
    out = _make_gather(npos, batch)(xt, w2)
    return out.reshape(npos, batch, DIM).transpose(1, 0, 2)
